# baseline (speedup 1.0000x reference)
import functools

import jax
import jax.numpy as jnp
from jax import lax
from jax.experimental import pallas as pl
from jax.experimental.pallas import tpu as pltpu
from jax.experimental.pallas import tpu_sc as plsc

N = 10000
D = 256
H = 128
E = 160000
NC = 2
NS = 16
C = 64
NR = 624
REM = N - NS * NR
EPP = 10240
NCH = EPP // C
NBUF = 5
NI = 10
GRP = 10

R = 1000
NB = N // R


def _sc_aggregate_body(y_st, eidx, eww, rst_st,
                       agg_sh, idxb, ewb, rows, isem, wsem, gsem, ssem):
    c = lax.axis_index("c")
    s = lax.axis_index("s")
    row0 = c * N + s * NR
    tbl_off = c * N

    def start_idx(k, bi):
        pltpu.async_copy(eidx.at[s * NCH + k], idxb.at[bi], isem.at[bi])
        pltpu.async_copy(eww.at[s * NCH + k], ewb.at[bi], wsem.at[bi])

    def wait_idx(k, bi):
        pltpu.make_async_copy(eidx.at[s * NCH + k], idxb.at[bi],
                              isem.at[bi]).wait()
        pltpu.make_async_copy(eww.at[s * NCH + k], ewb.at[bi],
                              wsem.at[bi]).wait()
        for g in range(C // 16):
            sl = pl.ds(g * 16, 16)
            idxb[bi, 0, sl] = idxb[bi, 0, sl] + tbl_off

    def start_gather(k, bi, br):
        pltpu.async_copy(y_st.at[idxb.at[bi, 0]], rows.at[br], gsem.at[br])

    def wait_gather(k, bi, br):
        pltpu.make_async_copy(y_st.at[idxb.at[bi, 0]], rows.at[br],
                              gsem.at[br]).wait()

    def do_scale(k, bi, br):
        @pl.loop(0, C // 16)
        def _scale(g):
            w16 = ewb[bi, pl.ds(g * 16, 16)]
            for e in range(16):
                w = w16[e]
                for j in range(H // 16):
                    sl = pl.ds(j * 16, 16)
                    rows[br, g * 16 + e, sl] = rows[br, g * 16 + e, sl] * w

    def start_scatter(k, bi, br):
        pltpu.async_copy(rows.at[br], agg_sh.at[idxb.at[bi, 1]], ssem.at[br],
                         add=True)

    def wait_scatter(k, bi, br):
        pltpu.make_async_copy(rows.at[br], agg_sh.at[idxb.at[bi, 1]],
                              ssem.at[br]).wait()

    def step(k, par, peeled):
        br = par % NBUF
        bi = par % NI
        wait_gather(k, bi, br)
        do_scale(k, bi, br)
        start_scatter(k, bi, br)

        bi5 = (par + 5) % NI
        br2 = (par + 2) % NBUF
        bi2 = (par + 2) % NI
        bi7 = (par + 7) % NI

        def _refill_idx():
            start_idx(k + 5, bi5)

        def _refill_gather():
            wait_scatter(k - 3, bi7, br2)
            wait_idx(k + 2, bi2)
            start_gather(k + 2, bi2, br2)

        def _first_gather():
            wait_idx(k + 2, bi2)
            start_gather(k + 2, bi2, br2)

        if peeled:
            _refill_idx()
            if par >= 3:
                _refill_gather()
            else:
                _first_gather()
        else:
            pl.when(k + 5 < NCH)(_refill_idx)
            pl.when(k + 2 < NCH)(_refill_gather)

    for k in range(5):
        start_idx(k, k)
    for k in range(2):
        wait_idx(k, k)
        start_gather(k, k, k)

    pltpu.sync_copy(y_st.at[pl.ds(row0, NR)], agg_sh.at[pl.ds(s * NR, NR)])

    @pl.when(s == NS - 1)
    def _seed_rem():
        pltpu.sync_copy(y_st.at[pl.ds(c * N + NS * NR, REM)],
                        agg_sh.at[pl.ds(NS * NR, REM)])

    plsc.subcore_barrier()

    for par in range(GRP):
        step(par, par, True)

    @pl.loop(1, NCH // GRP)
    def _grp(kk):
        for par in range(GRP):
            step(kk * GRP + par, par, False)

    for ch in range(NCH - 5, NCH):
        wait_scatter(ch, ch % NI, ch % NBUF)

    plsc.subcore_barrier()
    pltpu.sync_copy(agg_sh.at[pl.ds(s * NR, NR)], rst_st.at[pl.ds(row0, NR)])

    @pl.when(s == NS - 1)
    def _write_rem():
        pltpu.sync_copy(agg_sh.at[pl.ds(NS * NR, REM)],
                        rst_st.at[pl.ds(c * N + NS * NR, REM)])


@functools.cache
def _build_sc_aggregate():
    mesh = plsc.VectorSubcoreMesh(core_axis_name="c", subcore_axis_name="s",
                                  num_cores=NC, num_subcores=NS)
    return pl.kernel(
        _sc_aggregate_body,
        out_type=jax.ShapeDtypeStruct((NC * N, H), jnp.float32),
        mesh=mesh,
        scratch_types=[
            pltpu.VMEM_SHARED((N, H), jnp.float32),
            pltpu.VMEM((NI, 2, C), jnp.int32),
            pltpu.VMEM((NI, C), jnp.float32),
            pltpu.VMEM((NBUF, C, H), jnp.float32),
            pltpu.SemaphoreType.DMA((NI,)),
            pltpu.SemaphoreType.DMA((NI,)),
            pltpu.SemaphoreType.DMA((NBUF,)),
            pltpu.SemaphoreType.DMA((NBUF,)),
        ],
    )


def _tc_linear_body(lo, hi, wlo, whi, b, out):
    acc = jnp.dot(lo[...], wlo[...], preferred_element_type=jnp.float32)
    acc += jnp.dot(hi[...], whi[...], preferred_element_type=jnp.float32)
    out[...] = jnp.maximum(acc + b[...], 0.0)


def _tc_mean_body(lo, hi, wlo, whi, b, out):
    i = pl.program_id(1)
    acc = jnp.dot(lo[...], wlo[...], preferred_element_type=jnp.float32)
    acc += jnp.dot(hi[...], whi[...], preferred_element_type=jnp.float32)
    x2 = jnp.maximum(acc + b[...], 0.0)
    ssum = jnp.sum(x2, axis=0, keepdims=True)

    @pl.when(i == 0)
    def _():
        out[...] = jnp.zeros_like(out)

    out[...] += ssum

    @pl.when(i == NB - 1)
    def _():
        out[...] = out[...] * (1.0 / N)


_IN_SPECS = [
    pl.BlockSpec((R, H), lambda j, i: (i, 0)),
    pl.BlockSpec((R, H), lambda j, i: (NB + i, 0)),
    pl.BlockSpec((H, H), lambda j, i: (0, j)),
    pl.BlockSpec((H, H), lambda j, i: (1, j)),
    pl.BlockSpec((1, H), lambda j, i: (0, j)),
]


@jax.jit
def _tc_linear(rst_st, wt, b2):
    return pl.pallas_call(
        _tc_linear_body,
        grid=(2, NB),
        in_specs=_IN_SPECS,
        out_specs=pl.BlockSpec((R, H), lambda j, i: (j * NB + i, 0)),
        out_shape=jax.ShapeDtypeStruct((NC * N, H), jnp.float32),
    )(rst_st, rst_st, wt, wt, b2)


@jax.jit
def _tc_mean(rst_st, wt, b2):
    return pl.pallas_call(
        _tc_mean_body,
        grid=(2, NB),
        in_specs=_IN_SPECS,
        out_specs=pl.BlockSpec((1, H), lambda j, i: (0, j)),
        out_shape=jax.ShapeDtypeStruct((1, D), jnp.float32),
    )(rst_st, rst_st, wt, wt, b2)


def kernel(h, edge_index, edge_weight, W, b):
    pad = NS * EPP - E
    src = jnp.pad(edge_index[0].astype(jnp.int32).reshape(NS, E // NS),
                  ((0, 0), (0, pad // NS)))
    dst = jnp.pad(edge_index[1].astype(jnp.int32).reshape(NS, E // NS),
                  ((0, 0), (0, pad // NS)))
    eww = jnp.pad(edge_weight.astype(jnp.float32).reshape(NS, E // NS),
                  ((0, 0), (0, pad // NS))).reshape(NS * NCH, C)
    eidx = jnp.stack(
        [src.reshape(NS, NCH, C), dst.reshape(NS, NCH, C)],
        axis=2).reshape(NS * NCH, 2, C)
    h_st = jnp.concatenate([h[:, :H], h[:, H:]], axis=0)
    wt = W.T
    b2 = b.reshape(1, D)

    sc_aggregate = _build_sc_aggregate()
    rst1 = sc_aggregate(h_st, eidx, eww)
    x_st = _tc_linear(rst1, wt, b2)
    rst2 = sc_aggregate(x_st, eidx, eww)
    return _tc_mean(rst2, wt, b2)


# speedup vs baseline: 1.0565x; 1.0565x over previous
import functools

import jax
import jax.numpy as jnp
from jax import lax
from jax.experimental import pallas as pl
from jax.experimental.pallas import tpu as pltpu
from jax.experimental.pallas import tpu_sc as plsc

N = 10000
D = 256
H = 128
E = 160000
NC = 2
NS = 16
C = 64
NR = 624
REM = N - NS * NR
EPP = 10240
NCH = EPP // C
NBUF = 5
NI = 10
GRP = 10

R = 1000
NB = N // R


def _sc_aggregate_body(y_st, eidx, eww, rst_st,
                       agg_sh, idxb, ewb, rows, isem, wsem, gsem, ssem):
    c = lax.axis_index("c")
    s = lax.axis_index("s")
    row0 = c * N + s * NR
    tbl_off = c * N

    def start_idx(k, bi):
        pltpu.async_copy(eidx.at[s * NCH + k], idxb.at[bi], isem.at[bi])
        pltpu.async_copy(eww.at[s * NCH + k], ewb.at[bi], wsem.at[bi])

    def wait_idx(k, bi):
        pltpu.make_async_copy(eidx.at[s * NCH + k], idxb.at[bi],
                              isem.at[bi]).wait()
        pltpu.make_async_copy(eww.at[s * NCH + k], ewb.at[bi],
                              wsem.at[bi]).wait()
        for g in range(C // 16):
            sl = pl.ds(g * 16, 16)
            idxb[bi, 0, sl] = idxb[bi, 0, sl] + tbl_off

    def start_gather(k, bi, br):
        pltpu.async_copy(y_st.at[idxb.at[bi, 0]], rows.at[br], gsem.at[br])

    def wait_gather(k, bi, br):
        pltpu.make_async_copy(y_st.at[idxb.at[bi, 0]], rows.at[br],
                              gsem.at[br]).wait()

    def do_scale(k, bi, br):
        @pl.loop(0, C // 16)
        def _scale(g):
            w16 = ewb[bi, pl.ds(g * 16, 16)]
            for e in range(16):
                w = w16[e]
                for j in range(H // 16):
                    sl = pl.ds(j * 16, 16)
                    rows[br, g * 16 + e, sl] = rows[br, g * 16 + e, sl] * w

    def start_scatter(k, bi, br):
        pltpu.async_copy(rows.at[br], agg_sh.at[idxb.at[bi, 1]], ssem.at[br],
                         add=True)

    def wait_scatter(k, bi, br):
        pltpu.make_async_copy(rows.at[br], agg_sh.at[idxb.at[bi, 1]],
                              ssem.at[br]).wait()

    def step(k, par, peeled):
        br = par % NBUF
        bi = par % NI
        wait_gather(k, bi, br)
        do_scale(k, bi, br)
        start_scatter(k, bi, br)

        bi5 = (par + 5) % NI
        br3 = (par + 3) % NBUF
        bi3 = (par + 3) % NI
        bi8 = (par + 8) % NI

        def _refill_idx():
            start_idx(k + 5, bi5)

        def _refill_gather():
            wait_scatter(k - 2, bi8, br3)
            wait_idx(k + 3, bi3)
            start_gather(k + 3, bi3, br3)

        def _first_gather():
            wait_idx(k + 3, bi3)
            start_gather(k + 3, bi3, br3)

        if peeled:
            _refill_idx()
            if par >= 2:
                _refill_gather()
            else:
                _first_gather()
        else:
            pl.when(k + 5 < NCH)(_refill_idx)
            pl.when(k + 3 < NCH)(_refill_gather)

    for k in range(5):
        start_idx(k, k)
    for k in range(3):
        wait_idx(k, k)
        start_gather(k, k, k)

    pltpu.sync_copy(y_st.at[pl.ds(row0, NR)], agg_sh.at[pl.ds(s * NR, NR)])

    @pl.when(s == NS - 1)
    def _seed_rem():
        pltpu.sync_copy(y_st.at[pl.ds(c * N + NS * NR, REM)],
                        agg_sh.at[pl.ds(NS * NR, REM)])

    plsc.subcore_barrier()

    for par in range(GRP):
        step(par, par, True)

    @pl.loop(1, NCH // GRP)
    def _grp(kk):
        for par in range(GRP):
            step(kk * GRP + par, par, False)

    for ch in range(NCH - 5, NCH):
        wait_scatter(ch, ch % NI, ch % NBUF)

    plsc.subcore_barrier()
    pltpu.sync_copy(agg_sh.at[pl.ds(s * NR, NR)], rst_st.at[pl.ds(row0, NR)])

    @pl.when(s == NS - 1)
    def _write_rem():
        pltpu.sync_copy(agg_sh.at[pl.ds(NS * NR, REM)],
                        rst_st.at[pl.ds(c * N + NS * NR, REM)])


@functools.cache
def _build_sc_aggregate():
    mesh = plsc.VectorSubcoreMesh(core_axis_name="c", subcore_axis_name="s",
                                  num_cores=NC, num_subcores=NS)
    return pl.kernel(
        _sc_aggregate_body,
        out_type=jax.ShapeDtypeStruct((NC * N, H), jnp.float32),
        mesh=mesh,
        scratch_types=[
            pltpu.VMEM_SHARED((N, H), jnp.float32),
            pltpu.VMEM((NI, 2, C), jnp.int32),
            pltpu.VMEM((NI, C), jnp.float32),
            pltpu.VMEM((NBUF, C, H), jnp.float32),
            pltpu.SemaphoreType.DMA((NI,)),
            pltpu.SemaphoreType.DMA((NI,)),
            pltpu.SemaphoreType.DMA((NBUF,)),
            pltpu.SemaphoreType.DMA((NBUF,)),
        ],
    )


def _tc_linear_body(lo, hi, wlo, whi, b, out):
    acc = jnp.dot(lo[...], wlo[...], preferred_element_type=jnp.float32)
    acc += jnp.dot(hi[...], whi[...], preferred_element_type=jnp.float32)
    out[...] = jnp.maximum(acc + b[...], 0.0)


def _tc_mean_body(lo, hi, wlo, whi, b, out):
    i = pl.program_id(1)
    acc = jnp.dot(lo[...], wlo[...], preferred_element_type=jnp.float32)
    acc += jnp.dot(hi[...], whi[...], preferred_element_type=jnp.float32)
    x2 = jnp.maximum(acc + b[...], 0.0)
    ssum = jnp.sum(x2, axis=0, keepdims=True)

    @pl.when(i == 0)
    def _():
        out[...] = jnp.zeros_like(out)

    out[...] += ssum

    @pl.when(i == NB - 1)
    def _():
        out[...] = out[...] * (1.0 / N)


_IN_SPECS = [
    pl.BlockSpec((R, H), lambda j, i: (i, 0)),
    pl.BlockSpec((R, H), lambda j, i: (NB + i, 0)),
    pl.BlockSpec((H, H), lambda j, i: (0, j)),
    pl.BlockSpec((H, H), lambda j, i: (1, j)),
    pl.BlockSpec((1, H), lambda j, i: (0, j)),
]


@jax.jit
def _tc_linear(rst_st, wt, b2):
    return pl.pallas_call(
        _tc_linear_body,
        grid=(2, NB),
        in_specs=_IN_SPECS,
        out_specs=pl.BlockSpec((R, H), lambda j, i: (j * NB + i, 0)),
        out_shape=jax.ShapeDtypeStruct((NC * N, H), jnp.float32),
    )(rst_st, rst_st, wt, wt, b2)


@jax.jit
def _tc_mean(rst_st, wt, b2):
    return pl.pallas_call(
        _tc_mean_body,
        grid=(2, NB),
        in_specs=_IN_SPECS,
        out_specs=pl.BlockSpec((1, H), lambda j, i: (0, j)),
        out_shape=jax.ShapeDtypeStruct((1, D), jnp.float32),
    )(rst_st, rst_st, wt, wt, b2)


def kernel(h, edge_index, edge_weight, W, b):
    pad = NS * EPP - E
    src = jnp.pad(edge_index[0].astype(jnp.int32).reshape(NS, E // NS),
                  ((0, 0), (0, pad // NS)))
    dst = jnp.pad(edge_index[1].astype(jnp.int32).reshape(NS, E // NS),
                  ((0, 0), (0, pad // NS)))
    eww = jnp.pad(edge_weight.astype(jnp.float32).reshape(NS, E // NS),
                  ((0, 0), (0, pad // NS))).reshape(NS * NCH, C)
    eidx = jnp.stack(
        [src.reshape(NS, NCH, C), dst.reshape(NS, NCH, C)],
        axis=2).reshape(NS * NCH, 2, C)
    h_st = jnp.concatenate([h[:, :H], h[:, H:]], axis=0)
    wt = W.T
    b2 = b.reshape(1, D)

    sc_aggregate = _build_sc_aggregate()
    rst1 = sc_aggregate(h_st, eidx, eww)
    x_st = _tc_linear(rst1, wt, b2)
    rst2 = sc_aggregate(x_st, eidx, eww)
    return _tc_mean(rst2, wt, b2)
